# baseline (speedup 1.0000x reference)
import jax
import jax.numpy as jnp
from jax import lax
from jax.experimental import pallas as pl
from jax.experimental.pallas import tpu as pltpu
from jax.experimental.pallas import tpu_sc as plsc

B = 4096
NCTX = 10
NPOS = 20
NNEG = 20
D = 64
NV = 1000000
NW = 32
BPW = B // NW
C = 4
NCHUNK = BPW // C
CU = C * NCTX
CW = C * NPOS
SPAD = 48
D2 = 2 * D

_mesh = plsc.VectorSubcoreMesh(core_axis_name="c", subcore_axis_name="s")


def _treduce(vs, lane):
    vs = list(vs)
    if len(vs) == 8:
        vs = [v + v[lane ^ 8] for v in vs]
        dists = (4, 2, 1)
    else:
        dists = (8, 4, 2, 1)
    for d in dists:
        msk = (lane & d) == 0
        half = len(vs) // 2
        vs = [jnp.where(msk, vs[i], vs[i + half])
              + (jnp.where(msk, vs[i + half], vs[i]))[lane ^ d]
              for i in range(half)]
    return vs[0]


def _offsets(idx, base, n):
    dyn, stat = base
    lo = (stat // 16) * 16
    nvec = (stat + n - 1 - lo) // 16 + 1
    vecs = [(idx[pl.ds(dyn + lo + 16 * v, 16)] & 1) * D for v in range(nvec)]
    out = []
    for t in range(n):
        q, l = divmod(stat + t - lo, 16)
        out.append(vecs[q][l])
    return out


def _dots(acc, rows, offs, base, n):
    out = []
    for t in range(n):
        off = offs[t]
        v = acc[0] * rows[base + t, pl.ds(off, 16)]
        for k in range(1, 4):
            v = v + acc[k] * rows[base + t, pl.ds(off + k * 16, 16)]
        out.append(v)
    return out


def _halve(idx, half, n):
    def body(t, carry):
        half[pl.ds(t * 16, 16)] = idx[pl.ds(t * 16, 16)] >> 1
        return carry
    lax.fori_loop(0, n // 16, body, 0)


def _sc_body(pos_u_hbm, pos_w_hbm, neg_w_hbm, uw_hbm, ww_hbm, s_hbm,
             idx_u, idx_p, idx_n, half_u, half_p, half_n,
             rows_u, rows_p, rows_n, s_v, sem):
    wid = lax.axis_index("s") * 2 + lax.axis_index("c")
    lane = lax.iota(jnp.int32, 16)
    pltpu.sync_copy(pos_u_hbm.at[pl.ds(wid * (BPW * NCTX), BPW * NCTX)],
                    idx_u.at[pl.ds(0, BPW * NCTX)])
    pltpu.sync_copy(pos_w_hbm.at[pl.ds(wid * (BPW * NPOS), BPW * NPOS)], idx_p)
    pltpu.sync_copy(neg_w_hbm.at[pl.ds(wid * (BPW * NNEG), BPW * NNEG)], idx_n)
    _halve(idx_u, half_u, BPW * NCTX)
    _halve(idx_p, half_p, BPW * NPOS)
    _halve(idx_n, half_n, BPW * NNEG)

    def chunk(g, carry):
        cu = pltpu.async_copy(uw_hbm.at[half_u.at[pl.ds(g * CU, CU)]], rows_u, sem)
        cp = pltpu.async_copy(ww_hbm.at[half_p.at[pl.ds(g * CW, CW)]], rows_p, sem)
        cn = pltpu.async_copy(ww_hbm.at[half_n.at[pl.ds(g * CW, CW)]], rows_n, sem)
        cu.wait()
        cp.wait()
        cn.wait()

        for r in range(C):
            ub = r * NCTX
            uoffs = _offsets(idx_u, (g * CU, ub), NCTX)
            acc = [jnp.zeros((16,), jnp.float32) for _ in range(4)]
            for i in range(NCTX):
                off = uoffs[i]
                for k in range(4):
                    acc[k] = acc[k] + rows_u[ub + i, pl.ds(off + k * 16, 16)]
            wb = r * NPOS
            poffs = _offsets(idx_p, (g * CW, wb), NPOS)
            noffs = _offsets(idx_n, (g * CW, wb), NNEG)
            pos_main = _treduce(_dots(acc, rows_p, poffs, wb, 16), lane)
            neg_main = _treduce(_dots(acc, rows_n, noffs, wb, 16), lane)
            tail = _treduce(_dots(acc, rows_p, poffs[16:], wb + 16, 4)
                            + _dots(acc, rows_n, noffs[16:], wb + 16, 4),
                            lane)
            sb = g * (C * SPAD) + r * SPAD
            s_v[pl.ds(sb, 16)] = pos_main
            s_v[pl.ds(sb + 16, 16)] = neg_main
            s_v[pl.ds(sb + 32, 16)] = tail
        return carry

    lax.fori_loop(0, NCHUNK, chunk, 0)
    pltpu.sync_copy(s_v, s_hbm.at[pl.ds(wid * (BPW * SPAD), BPW * SPAD)])


_sc_call = pl.kernel(
    _sc_body,
    out_type=[jax.ShapeDtypeStruct((B * SPAD,), jnp.float32)],
    mesh=_mesh,
    scratch_types=[
        pltpu.VMEM((BPW * NCTX + 16,), jnp.int32),
        pltpu.VMEM((BPW * NPOS,), jnp.int32),
        pltpu.VMEM((BPW * NNEG,), jnp.int32),
        pltpu.VMEM((BPW * NCTX,), jnp.int32),
        pltpu.VMEM((BPW * NPOS,), jnp.int32),
        pltpu.VMEM((BPW * NNEG,), jnp.int32),
        pltpu.VMEM((CU, D2), jnp.float32),
        pltpu.VMEM((CW, D2), jnp.float32),
        pltpu.VMEM((CW, D2), jnp.float32),
        pltpu.VMEM((BPW * SPAD,), jnp.float32),
        pltpu.SemaphoreType.DMA,
    ],
)


def _log_sigmoid(x):
    return jnp.minimum(x, 0.0) - jnp.log(1.0 + jnp.exp(-jnp.abs(x)))


def _tc_body(s_ref, ip_ref, in_ref, out_ref):
    s = s_ref[...]
    sp = jnp.concatenate([s[:, 0:16], s[:, 32:36]], axis=1)
    sn = jnp.concatenate([s[:, 16:32], s[:, 36:40]], axis=1)
    pos = jnp.where(ip_ref[...] == 0, 0.0, _log_sigmoid(-sp))
    neg = jnp.where(in_ref[...] == 0, 0.0, _log_sigmoid(sn))
    out_ref[0, 0] = -(jnp.sum(pos) + jnp.sum(neg))


_tc_call = pl.pallas_call(
    _tc_body,
    out_shape=jax.ShapeDtypeStruct((1, 1), jnp.float32),
    out_specs=pl.BlockSpec(memory_space=pltpu.SMEM),
)


def kernel(pos_u, pos_w, neg_w, u_weight, w_weight):
    (s,) = _sc_call(pos_u.reshape(-1), pos_w.reshape(-1), neg_w.reshape(-1),
                    u_weight.reshape(NV // 2, D2), w_weight.reshape(NV // 2, D2))
    loss = _tc_call(s.reshape(B, SPAD), pos_w, neg_w)
    return loss[0, 0]


# speedup vs baseline: 1.0660x; 1.0660x over previous
import jax
import jax.numpy as jnp
from jax import lax
from jax.experimental import pallas as pl
from jax.experimental.pallas import tpu as pltpu
from jax.experimental.pallas import tpu_sc as plsc

B = 4096
NCTX = 10
NPOS = 20
NNEG = 20
D = 64
NV = 1000000
NW = 32
BPW = B // NW
C = 4
NCHUNK = BPW // C
CU = C * NCTX
CW = C * NPOS
SPAD = 48
D2 = 2 * D

_mesh = plsc.VectorSubcoreMesh(core_axis_name="c", subcore_axis_name="s")


def _treduce(vs, lane):
    vs = list(vs)
    if len(vs) == 8:
        vs = [v + v[lane ^ 8] for v in vs]
        dists = (4, 2, 1)
    else:
        dists = (8, 4, 2, 1)
    for d in dists:
        msk = (lane & d) == 0
        half = len(vs) // 2
        vs = [jnp.where(msk, vs[i], vs[i + half])
              + (jnp.where(msk, vs[i + half], vs[i]))[lane ^ d]
              for i in range(half)]
    return vs[0]


def _dots(acc, rows, base, n):
    out = []
    for t in range(n):
        v = acc[0] * rows[base + t, pl.ds(0, 16)]
        for k in range(1, 4):
            v = v + acc[k] * rows[base + t, pl.ds(k * 16, 16)]
        out.append(v)
    return out


def _sc_body(pos_u_hbm, pos_w_hbm, neg_w_hbm, uw_hbm, ww_hbm, s_hbm,
             idx_u, idx_p, idx_n,
             rows_u, rows_p, rows_n, s_v, sem):
    wid = lax.axis_index("s") * 2 + lax.axis_index("c")
    lane = lax.iota(jnp.int32, 16)
    pltpu.sync_copy(pos_u_hbm.at[pl.ds(wid * (BPW * NCTX), BPW * NCTX)], idx_u)
    pltpu.sync_copy(pos_w_hbm.at[pl.ds(wid * (BPW * NPOS), BPW * NPOS)], idx_p)
    pltpu.sync_copy(neg_w_hbm.at[pl.ds(wid * (BPW * NNEG), BPW * NNEG)], idx_n)

    def chunk(g, carry):
        cu = pltpu.async_copy(uw_hbm.at[idx_u.at[pl.ds(g * CU, CU)]], rows_u, sem)
        cp = pltpu.async_copy(ww_hbm.at[idx_p.at[pl.ds(g * CW, CW)]], rows_p, sem)
        cn = pltpu.async_copy(ww_hbm.at[idx_n.at[pl.ds(g * CW, CW)]], rows_n, sem)
        cu.wait()
        cp.wait()
        cn.wait()

        for r in range(C):
            ub = r * NCTX
            acc = [rows_u[ub, pl.ds(k * 16, 16)] for k in range(4)]
            for i in range(1, NCTX):
                for k in range(4):
                    acc[k] = acc[k] + rows_u[ub + i, pl.ds(k * 16, 16)]
            wb = r * NPOS
            pos_main = _treduce(_dots(acc, rows_p, wb, 16), lane)
            neg_main = _treduce(_dots(acc, rows_n, wb, 16), lane)
            tail = _treduce(_dots(acc, rows_p, wb + 16, 4)
                            + _dots(acc, rows_n, wb + 16, 4),
                            lane)
            sb = g * (C * SPAD) + r * SPAD
            s_v[pl.ds(sb, 16)] = pos_main
            s_v[pl.ds(sb + 16, 16)] = neg_main
            s_v[pl.ds(sb + 32, 16)] = tail
        return carry

    lax.fori_loop(0, NCHUNK, chunk, 0)
    pltpu.sync_copy(s_v, s_hbm.at[pl.ds(wid * (BPW * SPAD), BPW * SPAD)])


_sc_call = pl.kernel(
    _sc_body,
    out_type=[jax.ShapeDtypeStruct((B * SPAD,), jnp.float32)],
    mesh=_mesh,
    scratch_types=[
        pltpu.VMEM((BPW * NCTX,), jnp.int32),
        pltpu.VMEM((BPW * NPOS,), jnp.int32),
        pltpu.VMEM((BPW * NNEG,), jnp.int32),
        pltpu.VMEM((CU, D2), jnp.float32),
        pltpu.VMEM((CW, D2), jnp.float32),
        pltpu.VMEM((CW, D2), jnp.float32),
        pltpu.VMEM((BPW * SPAD,), jnp.float32),
        pltpu.SemaphoreType.DMA,
    ],
    compiler_params=pltpu.CompilerParams(use_tc_tiling_on_sc=True),
)


def _log_sigmoid(x):
    return jnp.minimum(x, 0.0) - jnp.log(1.0 + jnp.exp(-jnp.abs(x)))


def _tc_body(s_ref, ip_ref, in_ref, out_ref):
    s = s_ref[...]
    sp = jnp.concatenate([s[:, 0:16], s[:, 32:36]], axis=1)
    sn = jnp.concatenate([s[:, 16:32], s[:, 36:40]], axis=1)
    pos = jnp.where(ip_ref[...] == 0, 0.0, _log_sigmoid(-sp))
    neg = jnp.where(in_ref[...] == 0, 0.0, _log_sigmoid(sn))
    out_ref[0, 0] = -(jnp.sum(pos) + jnp.sum(neg))


_tc_call = pl.pallas_call(
    _tc_body,
    out_shape=jax.ShapeDtypeStruct((1, 1), jnp.float32),
    out_specs=pl.BlockSpec(memory_space=pltpu.SMEM),
)


def kernel(pos_u, pos_w, neg_w, u_weight, w_weight):
    up = jnp.pad(u_weight, ((0, 0), (0, D)))
    wp = jnp.pad(w_weight, ((0, 0), (0, D)))
    (s,) = _sc_call(pos_u.reshape(-1), pos_w.reshape(-1), neg_w.reshape(-1),
                    up, wp)
    loss = _tc_call(s.reshape(B, SPAD), pos_w, neg_w)
    return loss[0, 0]
